# Initial kernel scaffold; baseline (speedup 1.0000x reference)
import jax
import jax.numpy as jnp
from jax.experimental import pallas as pl


def kernel(idx, targets, table):
    raise NotImplementedError("write your pallas kernel here")


# speedup vs baseline: 1.6861x; 1.6861x over previous
import functools

import jax
import jax.numpy as jnp
from jax import lax
from jax.experimental import pallas as pl
from jax.experimental.pallas import tpu as pltpu
from jax.experimental.pallas import tpu_sc as plsc

V = 1000
N = 51200
NC, NS = 2, 16
NW = NC * NS
BPW = N // NW
CS = 80
NCHUNK = BPW // CS

_ROW_BLK = 200


def _lse_body(table_ref, lse_ref):
    x = table_ref[...]
    m = jnp.max(x, axis=1)
    s = jnp.sum(jnp.exp(x - m[:, None]), axis=1)
    lse_ref[...] = (m + jnp.log(s))[None, :]


def _lse_tc(table):
    return pl.pallas_call(
        _lse_body,
        out_shape=jax.ShapeDtypeStruct((1, V), jnp.float32),
    )(table)


def _gather_loss_body(idx_hbm, tgt_hbm, table_hbm, lse_hbm, out_hbm, part_hbm,
                      idx_v, tgt_v, lse_v, rows_v, acc_v, sem):
    wid = lax.axis_index("s") * NC + lax.axis_index("c")
    base = wid * BPW
    pltpu.sync_copy(idx_hbm.at[pl.ds(base, BPW)], idx_v)
    pltpu.sync_copy(tgt_hbm.at[pl.ds(base, BPW)], tgt_v)
    pltpu.sync_copy(lse_hbm, lse_v)

    def chunk_body(c, acc):
        off = pl.multiple_of(c * CS, CS)
        pltpu.async_copy(
            table_hbm.at[idx_v.at[pl.ds(off, CS)]], rows_v, sem).wait()
        pltpu.sync_copy(rows_v, out_hbm.at[pl.ds(base + off, CS)])

        def grp(j, acc):
            o16 = pl.multiple_of(off + j * 16, 16)
            rid = lax.iota(jnp.int32, 16) + j * 16
            t16 = tgt_v[pl.ds(o16, 16)]
            i16 = idx_v[pl.ds(o16, 16)]
            picked = plsc.load_gather(rows_v, [rid, t16])
            lse16 = plsc.load_gather(lse_v, [i16])
            return acc + (lse16 - picked)

        return lax.fori_loop(0, CS // 16, grp, acc, unroll=True)

    acc = lax.fori_loop(0, NCHUNK, chunk_body, jnp.zeros((16,), jnp.float32))
    acc_v[...] = acc
    pltpu.sync_copy(acc_v, part_hbm.at[pl.ds(wid * 16, 16)])


@functools.cache
def _gather_loss():
    return pl.kernel(
        _gather_loss_body,
        out_type=(
            jax.ShapeDtypeStruct((N, V), jnp.float32),
            jax.ShapeDtypeStruct((NW * 16,), jnp.float32),
        ),
        mesh=plsc.VectorSubcoreMesh(core_axis_name="c", subcore_axis_name="s"),
        scratch_types=[
            pltpu.VMEM((BPW,), jnp.int32),
            pltpu.VMEM((BPW,), jnp.int32),
            pltpu.VMEM((V,), jnp.float32),
            pltpu.VMEM((CS, V), jnp.float32),
            pltpu.VMEM((16,), jnp.float32),
            pltpu.SemaphoreType.DMA,
        ],
        compiler_params=pltpu.CompilerParams(use_tc_tiling_on_sc=False,
                                             needs_layout_passes=False),
    )


def _sum_body(p_ref, o_ref):
    o_ref[...] = jnp.sum(p_ref[...], keepdims=True) * (1.0 / N)


def _loss_tc(partials):
    return pl.pallas_call(
        _sum_body,
        out_shape=jax.ShapeDtypeStruct((1, 1), jnp.float32),
    )(partials)


def kernel(idx, targets, table):
    idx_f = idx.reshape(-1)
    tgt_f = targets.reshape(-1)
    lse = _lse_tc(table).reshape(-1)
    logits2d, partials = _gather_loss()(idx_f, tgt_f, table, lse)
    loss = _loss_tc(partials.reshape(1, -1))[0, 0]
    return logits2d, loss


# speedup vs baseline: 1.7143x; 1.0167x over previous
import functools

import jax
import jax.numpy as jnp
from jax import lax
from jax.experimental import pallas as pl
from jax.experimental.pallas import tpu as pltpu
from jax.experimental.pallas import tpu_sc as plsc

V = 1000
N = 51200
NC, NS = 2, 16
NW = NC * NS
BPW = N // NW
CS = 64
NCHUNK = BPW // CS


def _lse_body(table_ref, lse_ref):
    x = table_ref[...]
    m = jnp.max(x, axis=1)
    s = jnp.sum(jnp.exp(x - m[:, None]), axis=1)
    lse_ref[...] = (m + jnp.log(s))[None, :]


def _lse_tc(table):
    return pl.pallas_call(
        _lse_body,
        out_shape=jax.ShapeDtypeStruct((1, V), jnp.float32),
    )(table)


def _gather_loss_body(idx_hbm, tgt_hbm, table_hbm, lse_hbm, out_hbm, part_hbm,
                      lse_v, rows_a, rows_b, idx_a, idx_b, tgt_a, tgt_b,
                      acc_v, gsem_a, gsem_b, ssem_a, ssem_b, isem_a, isem_b):
    wid = lax.axis_index("s") * NC + lax.axis_index("c")
    base = wid * BPW
    rows = (rows_a, rows_b)
    idxs = (idx_a, idx_b)
    tgts = (tgt_a, tgt_b)
    gsem = (gsem_a, gsem_b)
    ssem = (ssem_a, ssem_b)
    isem = (isem_a, isem_b)

    pltpu.sync_copy(lse_hbm, lse_v)

    def fetch_meta(c, s):
        off = base + c * CS
        pltpu.async_copy(idx_hbm.at[pl.ds(off, CS)], idxs[s], isem[s])
        pltpu.async_copy(tgt_hbm.at[pl.ds(off, CS)], tgts[s], isem[s])

    def wait_meta(s):
        pltpu.make_async_copy(idx_hbm.at[pl.ds(0, CS)], idxs[s],
                              isem[s]).wait()
        pltpu.make_async_copy(tgt_hbm.at[pl.ds(0, CS)], tgts[s],
                              isem[s]).wait()

    def wait_gather(s):
        pltpu.make_async_copy(table_hbm.at[idxs[s]], rows[s], gsem[s]).wait()

    def wait_scatter(s):
        pltpu.make_async_copy(rows[s], out_hbm.at[pl.ds(0, CS)],
                              ssem[s]).wait()

    fetch_meta(0, 0)
    wait_meta(0)
    pltpu.async_copy(table_hbm.at[idxs[0]], rows[0], gsem[0])
    if NCHUNK > 1:
        fetch_meta(1, 1)

    acc = jnp.zeros((16,), jnp.float32)
    for c in range(NCHUNK):
        s = c % 2
        o = (c + 1) % 2
        if c + 1 < NCHUNK:
            if c >= 1:
                wait_scatter(o)
            wait_meta(o)
            pltpu.async_copy(table_hbm.at[idxs[o]], rows[o], gsem[o])
        wait_gather(s)
        pltpu.async_copy(rows[s], out_hbm.at[pl.ds(base + c * CS, CS)],
                         ssem[s])
        for j in range(CS // 16):
            rid = lax.iota(jnp.int32, 16) + j * 16
            t16 = tgts[s][pl.ds(j * 16, 16)]
            i16 = idxs[s][pl.ds(j * 16, 16)]
            picked = plsc.load_gather(rows[s], [rid, t16])
            lse16 = plsc.load_gather(lse_v, [i16])
            acc = acc + (lse16 - picked)
        if c + 2 < NCHUNK:
            fetch_meta(c + 2, s)

    wait_scatter((NCHUNK - 2) % 2)
    wait_scatter((NCHUNK - 1) % 2)
    acc_v[...] = acc
    pltpu.sync_copy(acc_v, part_hbm.at[pl.ds(wid * 16, 16)])


@functools.cache
def _gather_loss():
    return pl.kernel(
        _gather_loss_body,
        out_type=(
            jax.ShapeDtypeStruct((N, V), jnp.float32),
            jax.ShapeDtypeStruct((NW * 16,), jnp.float32),
        ),
        mesh=plsc.VectorSubcoreMesh(core_axis_name="c", subcore_axis_name="s"),
        scratch_types=[
            pltpu.VMEM((V,), jnp.float32),
            pltpu.VMEM((CS, V), jnp.float32),
            pltpu.VMEM((CS, V), jnp.float32),
            pltpu.VMEM((CS,), jnp.int32),
            pltpu.VMEM((CS,), jnp.int32),
            pltpu.VMEM((CS,), jnp.int32),
            pltpu.VMEM((CS,), jnp.int32),
            pltpu.VMEM((16,), jnp.float32),
            pltpu.SemaphoreType.DMA,
            pltpu.SemaphoreType.DMA,
            pltpu.SemaphoreType.DMA,
            pltpu.SemaphoreType.DMA,
            pltpu.SemaphoreType.DMA,
            pltpu.SemaphoreType.DMA,
        ],
        compiler_params=pltpu.CompilerParams(use_tc_tiling_on_sc=False,
                                             needs_layout_passes=False),
    )


def _sum_body(p_ref, o_ref):
    o_ref[...] = jnp.sum(p_ref[...], keepdims=True) * (1.0 / N)


def _loss_tc(partials):
    return pl.pallas_call(
        _sum_body,
        out_shape=jax.ShapeDtypeStruct((1, 1), jnp.float32),
    )(partials)


def kernel(idx, targets, table):
    idx_f = idx.reshape(-1)
    tgt_f = targets.reshape(-1)
    lse = _lse_tc(table).reshape(-1)
    logits2d, partials = _gather_loss()(idx_f, tgt_f, table, lse)
    loss = _loss_tc(partials.reshape(1, -1))[0, 0]
    return logits2d, loss
